# baseline (speedup 1.0000x reference)
import functools

import jax
import jax.numpy as jnp
from jax import lax
from jax.experimental import pallas as pl
from jax.experimental.pallas import tpu as pltpu
from jax.experimental.pallas import tpu_sc as plsc

LN_EPS = 1e-5

NC = 2
NS = 16
NW = NC * NS
LANES = 16


def _rsqrt_vec(x):
    i = plsc.bitcast(x, jnp.int32)
    i = jnp.int32(0x5F3759DF) - (i >> 1)
    y = plsc.bitcast(i, jnp.float32)
    half = x * 0.5
    for _ in range(3):
        y = y * (1.5 - half * y * y)
    return y


def _make_sc_call(n_seqs, vocab, d, seq_len):
    assert d == 4 * LANES
    assert n_seqs % NW == 0
    seqs_per_w = n_seqs // NW
    assert seqs_per_w % 2 == 0
    nj = d // LANES
    n_tok = n_seqs * seq_len

    mesh = plsc.VectorSubcoreMesh(
        core_axis_name="c", subcore_axis_name="s",
        num_cores=NC, num_subcores=NS,
    )

    def body(ids_hbm, idsh_hbm, tok2_hbm, pos_t_hbm, g_hbm, b_hbm, out_hbm,
             ids0, ids1, idsh0, idsh1, rows0, rows1, out0, out1,
             pos_t_v, g_v, b_v,
             sem_g0, sem_g1, sem_o0, sem_o1):
        wid = lax.axis_index("s") * NC + lax.axis_index("c")
        seq_base = wid * seqs_per_w
        tok_base = seq_base * seq_len

        pltpu.sync_copy(pos_t_hbm, pos_t_v)
        pltpu.sync_copy(g_hbm, g_v)
        pltpu.sync_copy(b_hbm, b_v)
        gs = [g_v[pl.ds(LANES * j, LANES)] for j in range(nj)]
        bs = [b_v[pl.ds(LANES * j, LANES)] for j in range(nj)]

        dim_base = lax.iota(jnp.int32, LANES)
        dim_vecs = [dim_base + LANES * j for j in range(nj)]

        idss = [ids0, ids1]
        idshs = [idsh0, idsh1]
        rows = [rows0, rows1]
        outs = [out0, out1]
        sems_g = [sem_g0, sem_g1]
        sems_o = [sem_o0, sem_o1]

        def gather(c, b):
            base = tok_base + c * seq_len
            pltpu.sync_copy(ids_hbm.at[pl.ds(base, seq_len)],
                            idss[b].at[pl.ds(0, seq_len)])
            pltpu.sync_copy(idsh_hbm.at[pl.ds(base, seq_len)], idshs[b])
            pltpu.async_copy(tok2_hbm.at[idshs[b]], rows[b], sems_g[b])

        def wait_gather(b):
            pltpu.make_async_copy(
                tok2_hbm.at[idshs[b]], rows[b], sems_g[b]).wait()

        def put(c, b):
            pltpu.async_copy(outs[b], out_hbm.at[seq_base + c], sems_o[b])

        def wait_put(b):
            pltpu.make_async_copy(outs[b], out_hbm.at[0], sems_o[b]).wait()

        def compute(b):
            ids_v = idss[b]
            rows_v = rows[b]
            out_v = outs[b]

            @plsc.parallel_loop(0, seq_len, unroll=4)
            def token_body(i):
                hv = ids_v[pl.ds(i, LANES)]
                hoff = (hv[0] & 1) * d
                col = jnp.full((LANES,), i, dtype=jnp.int32)
                e = [rows_v[i, pl.ds(hoff + LANES * j, LANES)]
                     + plsc.load_gather(pos_t_v, [dim_vecs[j], col])
                     for j in range(nj)]
                t = (e[0] + e[1]) + (e[2] + e[3])
                sq = [ej * ej for ej in e]
                ts = (sq[0] + sq[1]) + (sq[2] + sq[3])
                s = jnp.broadcast_to(jnp.sum(t), (LANES,))
                ss = jnp.broadcast_to(jnp.sum(ts), (LANES,))
                mean = s * (1.0 / d)
                var = ss * (1.0 / d) - mean * mean
                rinv = _rsqrt_vec(var + LN_EPS)
                for j in range(nj):
                    out_v[i, pl.ds(LANES * j, LANES)] = (
                        (e[j] - mean) * (rinv * gs[j]) + bs[j])

        gather(0, 0)
        gather(1, 1)

        def pair_body(i, carry):
            c0 = 2 * i
            for b in range(2):
                c = c0 + b
                wait_gather(b)

                @pl.when(c >= 2)
                def _():
                    wait_put(b)

                compute(b)
                put(c, b)

                @pl.when(c + 2 < seqs_per_w)
                def _():
                    gather(c + 2, b)
            return carry

        lax.fori_loop(0, seqs_per_w // 2, pair_body, 0)
        wait_put(0)
        wait_put(1)

    return pl.kernel(
        body,
        out_type=jax.ShapeDtypeStruct((n_seqs, seq_len, d), jnp.float32),
        mesh=mesh,
        compiler_params=pltpu.CompilerParams(
            needs_layout_passes=False, use_tc_tiling_on_sc=True),
        scratch_types=[
            pltpu.VMEM((seq_len + LANES,), jnp.int32),
            pltpu.VMEM((seq_len + LANES,), jnp.int32),
            pltpu.VMEM((seq_len,), jnp.int32),
            pltpu.VMEM((seq_len,), jnp.int32),
            pltpu.VMEM((seq_len, 2 * d), jnp.float32),
            pltpu.VMEM((seq_len, 2 * d), jnp.float32),
            pltpu.VMEM((seq_len, d), jnp.float32),
            pltpu.VMEM((seq_len, d), jnp.float32),
            pltpu.VMEM((d, seq_len), jnp.float32),
            pltpu.VMEM((d,), jnp.float32),
            pltpu.VMEM((d,), jnp.float32),
            pltpu.SemaphoreType.DMA,
            pltpu.SemaphoreType.DMA,
            pltpu.SemaphoreType.DMA,
            pltpu.SemaphoreType.DMA,
        ],
    )


def kernel(token_ids, token_table, pos_table, ln_gamma, ln_beta):
    batch, seq_len = token_ids.shape
    vocab, d = token_table.shape
    ids32 = token_ids.astype(jnp.int32)
    ids = ids32.reshape(-1)
    idsh = (ids32 >> 1).reshape(-1)
    table2 = token_table.reshape(vocab // 2, 2 * d)
    call = _make_sc_call(batch, vocab, d, seq_len)
    return call(ids, idsh, table2, pos_table.T, ln_gamma, ln_beta)


# speedup vs baseline: 1.2116x; 1.2116x over previous
import functools

import jax
import jax.numpy as jnp
from jax import lax
from jax.experimental import pallas as pl
from jax.experimental.pallas import tpu as pltpu
from jax.experimental.pallas import tpu_sc as plsc

LN_EPS = 1e-5

NC = 2
NS = 16
NW = NC * NS
LANES = 16


def _rsqrt_vec(x):
    i = plsc.bitcast(x, jnp.int32)
    i = jnp.int32(0x5F3759DF) - (i >> 1)
    y = plsc.bitcast(i, jnp.float32)
    half = x * 0.5
    for _ in range(3):
        y = y * (1.5 - half * y * y)
    return y


def _make_sc_call(batch, vocab, d, seq_len):
    assert d == 4 * LANES
    assert batch % (NW * 2 * LANES) == 0
    bpw = batch // NW
    assert bpw == 2 * d
    nj = d // LANES
    n_tok = batch * seq_len
    chunk_f = (d // 8) * 8 * bpw
    s_stride = (d // 8) * 8 * batch

    mesh = plsc.VectorSubcoreMesh(
        core_axis_name="c", subcore_axis_name="s",
        num_cores=NC, num_subcores=NS,
    )

    def body(idsp_hbm, tok_hbm, pos_t_hbm, g_hbm, b_hbm, out_hbm,
             idx0, idx1, rows0, rows1, out0, out1, pos_t_v, g_v, b_v,
             sem_g0, sem_g1, sem_o0, sem_o1):
        wid = lax.axis_index("s") * NC + lax.axis_index("c")
        idx_base = wid * seq_len * bpw
        out_base = wid * bpw * 8

        pltpu.sync_copy(pos_t_hbm, pos_t_v)
        pltpu.sync_copy(g_hbm, g_v)
        pltpu.sync_copy(b_hbm, b_v)
        gs = [g_v[pl.ds(LANES * j, LANES)] for j in range(nj)]
        bs = [b_v[pl.ds(LANES * j, LANES)] for j in range(nj)]

        dim_base = lax.iota(jnp.int32, LANES)
        dim_vecs = [dim_base + LANES * j for j in range(nj)]
        scat = [((dim_vecs[j] >> 3) << 10) + ((dim_vecs[j] & 7) << 7)
                for j in range(nj)]

        idxs = [idx0, idx1]
        rows = [rows0, rows1]
        outs = [out0, out1]
        sems_g = [sem_g0, sem_g1]
        sems_o = [sem_o0, sem_o1]

        def gather(c, b):
            pltpu.sync_copy(idsp_hbm.at[pl.ds(idx_base + c * bpw, bpw)],
                            idxs[b])
            pltpu.async_copy(tok_hbm.at[idxs[b]], rows[b], sems_g[b])

        def wait_gather(b):
            pltpu.make_async_copy(
                tok_hbm.at[idxs[b]], rows[b], sems_g[b]).wait()

        def put(c, b):
            base = c * s_stride + out_base
            for j in range(8):
                pltpu.async_copy(
                    outs[b].at[pl.ds(j * bpw * 8, bpw * 8)],
                    out_hbm.at[pl.ds(base + j * 8 * batch, bpw * 8)],
                    sems_o[b])

        def wait_put(b):
            pltpu.make_async_copy(
                outs[b], out_hbm.at[pl.ds(0, chunk_f)], sems_o[b]).wait()

        def compute(s, b):
            rows_v = rows[b]
            out_v = outs[b]
            col = jnp.full((LANES,), s, dtype=jnp.int32)
            ps = [plsc.load_gather(pos_t_v, [dim_vecs[j], col])
                  for j in range(nj)]

            @plsc.parallel_loop(0, bpw, unroll=4)
            def token_body(i):
                e = [rows_v[i, pl.ds(LANES * j, LANES)] + ps[j]
                     for j in range(nj)]
                t = (e[0] + e[1]) + (e[2] + e[3])
                sq = [ej * ej for ej in e]
                ts = (sq[0] + sq[1]) + (sq[2] + sq[3])
                sm = jnp.broadcast_to(jnp.sum(t), (LANES,))
                ss = jnp.broadcast_to(jnp.sum(ts), (LANES,))
                mean = sm * (1.0 / d)
                var = ss * (1.0 / d) - mean * mean
                rinv = _rsqrt_vec(var + LN_EPS)
                iv = jnp.full((LANES,), i, dtype=jnp.int32)
                for j in range(nj):
                    o = (e[j] - mean) * (rinv * gs[j]) + bs[j]
                    plsc.store_scatter(out_v, [scat[j] + iv], o)

        gather(0, 0)
        gather(1, 1)

        def pair_body(ii, carry):
            c0 = 2 * ii
            for b in range(2):
                c = c0 + b
                wait_gather(b)

                @pl.when(c >= 2)
                def _():
                    wait_put(b)

                compute(c, b)
                put(c, b)

                @pl.when(c + 2 < seq_len)
                def _():
                    gather(c + 2, b)
            return carry

        lax.fori_loop(0, seq_len // 2, pair_body, 0)
        wait_put(0)
        wait_put(1)

    return pl.kernel(
        body,
        out_type=jax.ShapeDtypeStruct((n_tok * d,), jnp.float32),
        mesh=mesh,
        compiler_params=pltpu.CompilerParams(
            needs_layout_passes=False, use_tc_tiling_on_sc=False),
        scratch_types=[
            pltpu.VMEM((bpw,), jnp.int32),
            pltpu.VMEM((bpw,), jnp.int32),
            pltpu.VMEM((bpw, d), jnp.float32),
            pltpu.VMEM((bpw, d), jnp.float32),
            pltpu.VMEM((chunk_f,), jnp.float32),
            pltpu.VMEM((chunk_f,), jnp.float32),
            pltpu.VMEM((d, seq_len), jnp.float32),
            pltpu.VMEM((d,), jnp.float32),
            pltpu.VMEM((d,), jnp.float32),
            pltpu.SemaphoreType.DMA,
            pltpu.SemaphoreType.DMA,
            pltpu.SemaphoreType.DMA,
            pltpu.SemaphoreType.DMA,
        ],
    )


def kernel(token_ids, token_table, pos_table, ln_gamma, ln_beta):
    batch, seq_len = token_ids.shape
    vocab, d = token_table.shape
    ids32 = token_ids.astype(jnp.int32)
    idsp = (ids32.T.reshape(seq_len, NW, batch // NW)
            .transpose(1, 0, 2).reshape(-1))
    call = _make_sc_call(batch, vocab, d, seq_len)
    flat = call(idsp, token_table, pos_table.T, ln_gamma, ln_beta)
    out = (flat.reshape(seq_len, d // 8, NW, 8, batch // NW)
           .transpose(2, 4, 0, 1, 3).reshape(batch, seq_len, d))
    return out


# speedup vs baseline: 1.2914x; 1.0659x over previous
import functools

import jax
import jax.numpy as jnp
from jax import lax
from jax.experimental import pallas as pl
from jax.experimental.pallas import tpu as pltpu
from jax.experimental.pallas import tpu_sc as plsc

LN_EPS = 1e-5

NC = 2
NS = 16
NW = NC * NS
LANES = 16


def _rsqrt_vec(x):
    i = plsc.bitcast(x, jnp.int32)
    i = jnp.int32(0x5F3759DF) - (i >> 1)
    y = plsc.bitcast(i, jnp.float32)
    half = x * 0.5
    for _ in range(3):
        y = y * (1.5 - half * y * y)
    return y


def _make_sc_call(batch, vocab, d, seq_len):
    assert d == 4 * LANES
    bpw = batch // NW
    assert bpw == 2 * d
    nj = d // LANES
    n_tok = batch * seq_len
    tpw = bpw * seq_len
    run = bpw * 8

    mesh = plsc.VectorSubcoreMesh(
        core_axis_name="c", subcore_axis_name="s",
        num_cores=NC, num_subcores=NS,
    )

    def body(ids_hbm, tok_hbm, pos_t_hbm, g_hbm, b_hbm, out_hbm,
             ids_blk, idx_all, rows0, rows1, out0, out1, pos_t_v, g_v, b_v,
             sem_g0, sem_g1, sem_o0, sem_o1):
        wid = lax.axis_index("s") * NC + lax.axis_index("c")

        pltpu.sync_copy(pos_t_hbm, pos_t_v)
        pltpu.sync_copy(g_hbm, g_v)
        pltpu.sync_copy(b_hbm, b_v)
        pltpu.sync_copy(ids_hbm.at[pl.ds(wid * tpw, tpw)], ids_blk)
        gs = [g_v[pl.ds(LANES * j, LANES)] for j in range(nj)]
        bs = [b_v[pl.ds(LANES * j, LANES)] for j in range(nj)]

        dim_base = lax.iota(jnp.int32, LANES)
        dim_vecs = [dim_base + LANES * j for j in range(nj)]
        scat_rows = [dim_vecs[j] >> 3 for j in range(nj)]
        scat_cols = [(dim_vecs[j] & 7) << 7 for j in range(nj)]

        bvecs = [(dim_base + LANES * g) * seq_len for g in range(bpw // LANES)]

        @plsc.parallel_loop(0, seq_len, unroll=2)
        def tr_body(s):
            for g in range(bpw // LANES):
                v = plsc.load_gather(ids_blk, [bvecs[g] + s])
                idx_all[s, pl.ds(LANES * g, LANES)] = v

        rows = [rows0, rows1]
        outs = [out0, out1]
        sems_g = [sem_g0, sem_g1]
        sems_o = [sem_o0, sem_o1]

        def gather(c, b):
            pltpu.async_copy(tok_hbm.at[idx_all.at[c]], rows[b], sems_g[b])

        def wait_gather(b):
            pltpu.make_async_copy(
                tok_hbm.at[idx_all.at[0]], rows[b], sems_g[b]).wait()

        s_stride = 8 * NW * run

        def put(c, b):
            base = c * s_stride + wid * run
            for j in range(8):
                pltpu.async_copy(
                    outs[b].at[j],
                    out_hbm.at[pl.ds(base + j * NW * run, run)],
                    sems_o[b])

        def wait_put(b):
            pltpu.make_async_copy(
                outs[b], out_hbm.at[pl.ds(0, 8 * run)], sems_o[b]).wait()

        def compute(s, b):
            rows_v = rows[b]
            out_v = outs[b]
            col = jnp.full((LANES,), s, dtype=jnp.int32)
            ps = [plsc.load_gather(pos_t_v, [dim_vecs[j], col])
                  for j in range(nj)]

            @plsc.parallel_loop(0, bpw, unroll=4)
            def token_body(i):
                e = [rows_v[i, pl.ds(LANES * j, LANES)] + ps[j]
                     for j in range(nj)]
                t = (e[0] + e[1]) + (e[2] + e[3])
                sq = [ej * ej for ej in e]
                ts = (sq[0] + sq[1]) + (sq[2] + sq[3])
                sm = jnp.broadcast_to(jnp.sum(t), (LANES,))
                ss = jnp.broadcast_to(jnp.sum(ts), (LANES,))
                mean = sm * (1.0 / d)
                var = ss * (1.0 / d) - mean * mean
                rinv = _rsqrt_vec(var + LN_EPS)
                iv = jnp.full((LANES,), i, dtype=jnp.int32)
                for j in range(nj):
                    o = (e[j] - mean) * (rinv * gs[j]) + bs[j]
                    plsc.store_scatter(out_v, [scat_rows[j], scat_cols[j] + iv], o)

        gather(0, 0)
        gather(1, 1)

        def pair_body(ii, carry):
            c0 = 2 * ii
            for b in range(2):
                c = c0 + b
                wait_gather(b)

                @pl.when(c >= 2)
                def _():
                    wait_put(b)

                compute(c, b)
                put(c, b)

                @pl.when(c + 2 < seq_len)
                def _():
                    gather(c + 2, b)
            return carry

        lax.fori_loop(0, seq_len // 2, pair_body, 0)
        wait_put(0)
        wait_put(1)

    return pl.kernel(
        body,
        out_type=jax.ShapeDtypeStruct((n_tok * d,), jnp.float32),
        mesh=mesh,
        compiler_params=pltpu.CompilerParams(
            needs_layout_passes=False, use_tc_tiling_on_sc=False),
        scratch_types=[
            pltpu.VMEM((tpw,), jnp.int32),
            pltpu.VMEM((seq_len, bpw), jnp.int32),
            pltpu.VMEM((bpw, d), jnp.float32),
            pltpu.VMEM((bpw, d), jnp.float32),
            pltpu.VMEM((8, run), jnp.float32),
            pltpu.VMEM((8, run), jnp.float32),
            pltpu.VMEM((d, seq_len), jnp.float32),
            pltpu.VMEM((d,), jnp.float32),
            pltpu.VMEM((d,), jnp.float32),
            pltpu.SemaphoreType.DMA,
            pltpu.SemaphoreType.DMA,
            pltpu.SemaphoreType.DMA,
            pltpu.SemaphoreType.DMA,
        ],
    )


def kernel(token_ids, token_table, pos_table, ln_gamma, ln_beta):
    batch, seq_len = token_ids.shape
    vocab, d = token_table.shape
    ids = token_ids.astype(jnp.int32).reshape(-1)
    call = _make_sc_call(batch, vocab, d, seq_len)
    flat = call(ids, token_table, pos_table.T, ln_gamma, ln_beta)
    out = (flat.reshape(seq_len, d // 8, NW, 8, batch // NW)
           .transpose(2, 4, 0, 1, 3).reshape(batch, seq_len, d))
    return out


# speedup vs baseline: 1.5822x; 1.2251x over previous
import functools

import jax
import jax.numpy as jnp
from jax import lax
from jax.experimental import layout as jex_layout
from jax.experimental import pallas as pl
from jax.experimental.pallas import tpu as pltpu
from jax.experimental.pallas import tpu_sc as plsc

LN_EPS = 1e-5

NC = 2
NS = 16
NW = NC * NS
LANES = 16


def _rsqrt_vec(x):
    i = plsc.bitcast(x, jnp.int32)
    i = jnp.int32(0x5F3759DF) - (i >> 1)
    y = plsc.bitcast(i, jnp.float32)
    half = x * 0.5
    for _ in range(3):
        y = y * (1.5 - half * y * y)
    return y


def _make_sc_call(n_seqs, vocab, d, seq_len):
    assert d == 4 * LANES
    assert n_seqs % NW == 0
    seqs_per_w = n_seqs // NW
    assert seqs_per_w % 2 == 0
    nj = d // LANES

    mesh = plsc.VectorSubcoreMesh(
        core_axis_name="c", subcore_axis_name="s",
        num_cores=NC, num_subcores=NS,
    )

    def body(ids_hbm, tok_hbm, pos_hbm, g_hbm, b_hbm, out_hbm,
             idx_all, rows0, rows1, out0, out1, pos_v, g_v, b_v,
             sem_g0, sem_g1, sem_o0, sem_o1):
        wid = lax.axis_index("s") * NC + lax.axis_index("c")
        seq_base = wid * seqs_per_w

        pltpu.sync_copy(pos_hbm, pos_v)
        pltpu.sync_copy(g_hbm, g_v)
        pltpu.sync_copy(b_hbm, b_v)
        pltpu.sync_copy(ids_hbm.at[pl.ds(seq_base, seqs_per_w)], idx_all)
        gs = [g_v[pl.ds(LANES * j, LANES)] for j in range(nj)]
        bs = [b_v[pl.ds(LANES * j, LANES)] for j in range(nj)]

        rows = [rows0, rows1]
        outs = [out0, out1]
        sems_g = [sem_g0, sem_g1]
        sems_o = [sem_o0, sem_o1]

        def gather(c, b):
            pltpu.async_copy(tok_hbm.at[idx_all.at[c]], rows[b], sems_g[b])

        def wait_gather(b):
            pltpu.make_async_copy(
                tok_hbm.at[idx_all.at[0]], rows[b], sems_g[b]).wait()

        def put(c, b):
            pltpu.async_copy(
                outs[b], out_hbm.at[pl.ds((seq_base + c) * seq_len, seq_len)],
                sems_o[b])

        def wait_put(b):
            pltpu.make_async_copy(
                outs[b], out_hbm.at[pl.ds(0, seq_len)], sems_o[b]).wait()

        def compute(b):
            rows_v = rows[b]
            out_v = outs[b]

            @plsc.parallel_loop(0, seq_len, unroll=4)
            def token_body(i):
                e = [rows_v[i, pl.ds(LANES * j, LANES)]
                     + pos_v[i, pl.ds(LANES * j, LANES)]
                     for j in range(nj)]
                t = (e[0] + e[1]) + (e[2] + e[3])
                sq = [ej * ej for ej in e]
                ts = (sq[0] + sq[1]) + (sq[2] + sq[3])
                s = jnp.broadcast_to(jnp.sum(t), (LANES,))
                ss = jnp.broadcast_to(jnp.sum(ts), (LANES,))
                mean = s * (1.0 / d)
                var = ss * (1.0 / d) - mean * mean
                rinv = _rsqrt_vec(var + LN_EPS)
                for j in range(nj):
                    out_v[i, pl.ds(LANES * j, LANES)] = (
                        (e[j] - mean) * (rinv * gs[j]) + bs[j])

        gather(0, 0)
        gather(1, 1)

        def pair_body(i, carry):
            c0 = 2 * i
            for b in range(2):
                c = c0 + b
                wait_gather(b)

                @pl.when(c >= 2)
                def _():
                    wait_put(b)

                compute(b)
                put(c, b)

                @pl.when(c + 2 < seqs_per_w)
                def _():
                    gather(c + 2, b)
            return carry

        lax.fori_loop(0, seqs_per_w // 2, pair_body, 0)
        wait_put(0)
        wait_put(1)

    return pl.kernel(
        body,
        out_type=jax.ShapeDtypeStruct((n_seqs * seq_len, d), jnp.float32),
        mesh=mesh,
        compiler_params=pltpu.CompilerParams(
            needs_layout_passes=False, use_tc_tiling_on_sc=False,
            skip_device_barrier=True),
        scratch_types=[
            pltpu.VMEM((seqs_per_w, seq_len), jnp.int32),
            pltpu.VMEM((seq_len, d), jnp.float32),
            pltpu.VMEM((seq_len, d), jnp.float32),
            pltpu.VMEM((seq_len, d), jnp.float32),
            pltpu.VMEM((seq_len, d), jnp.float32),
            pltpu.VMEM((seq_len, d), jnp.float32),
            pltpu.VMEM((d,), jnp.float32),
            pltpu.VMEM((d,), jnp.float32),
            pltpu.SemaphoreType.DMA,
            pltpu.SemaphoreType.DMA,
            pltpu.SemaphoreType.DMA,
            pltpu.SemaphoreType.DMA,
        ],
    )


def _impl(token_ids, token_table, pos_table, ln_gamma, ln_beta):
    batch, seq_len = token_ids.shape
    vocab, d = token_table.shape
    n_tokens = batch * seq_len
    ids = token_ids.reshape(n_tokens // seq_len, seq_len).astype(jnp.int32)
    call = _make_sc_call(n_tokens // seq_len, vocab, d, seq_len)
    out = call(ids, token_table, pos_table, ln_gamma, ln_beta)
    return out.reshape(batch, seq_len, d)


@functools.lru_cache(maxsize=None)
def _jitted(sharding):
    fmt = jex_layout.Format(
        jex_layout.Layout(major_to_minor=(0, 1, 2)), sharding)
    return functools.partial(jax.jit, out_shardings=fmt)(_impl)


def kernel(token_ids, token_table, pos_table, ln_gamma, ln_beta):
    sharding = getattr(token_ids, "sharding", None)
    if sharding is None or isinstance(token_ids, jax.core.Tracer):
        return _impl(token_ids, token_table, pos_table, ln_gamma, ln_beta)
    return _jitted(sharding)(
        token_ids, token_table, pos_table, ln_gamma, ln_beta)
